# baseline (speedup 1.0000x reference)
import functools

import jax
import jax.numpy as jnp
from jax import lax
from jax.experimental import pallas as pl
from jax.experimental.pallas import tpu as pltpu
from jax.experimental.pallas import tpu_sc as plsc

D = 64
L = 16
NC, NS = 2, 16
NW = NC * NS
C = 128


def _lane_shuffle(v, idx):
    return lax.gather(
        v, idx[:, None],
        lax.GatherDimensionNumbers(
            offset_dims=(), collapsed_slice_dims=(0,), start_index_map=(0,)),
        (1,), mode=lax.GatherScatterMode.PROMISE_IN_BOUNDS)


def _lane_sum(v):
    lanes = lax.iota(jnp.int32, L)
    for k in (8, 4, 2, 1):
        v = v + _lane_shuffle(v, lanes ^ jnp.int32(k))
    return v


def _rsqrt_vec(x):
    i = lax.bitcast_convert_type(x, jnp.int32)
    i = jnp.int32(0x5F3759DF) - lax.shift_right_arithmetic(i, jnp.int32(1))
    y = lax.bitcast_convert_type(i, jnp.float32)
    for _ in range(3):
        y = y * (jnp.float32(1.5) - jnp.float32(0.5) * x * y * y)
    return y


def _make_sc_call(n_rows):
    assert n_rows % (NW * C) == 0
    rows_per_w = n_rows // NW
    n_chunks = rows_per_w // C
    mesh = plsc.VectorSubcoreMesh(core_axis_name="c", subcore_axis_name="s")

    @functools.partial(
        pl.kernel,
        mesh=mesh,
        compiler_params=pltpu.CompilerParams(use_tc_tiling_on_sc=False),
        out_type=jax.ShapeDtypeStruct((n_rows, D), jnp.float32),
        scratch_types=[
            pltpu.VMEM((C,), jnp.int32),
            pltpu.VMEM((C, D), jnp.float32),
            pltpu.VMEM((D,), jnp.float32),
            pltpu.VMEM((D,), jnp.float32),
            pltpu.SemaphoreType.DMA,
        ],
    )
    def sc_fn(idx_hbm, table_hbm, gamma_hbm, beta_hbm, out_hbm,
              idx_v, rows_v, gv, bv, sem):
        wid = lax.axis_index("s") * NC + lax.axis_index("c")
        w_base = wid * rows_per_w

        pltpu.sync_copy(gamma_hbm, gv)
        pltpu.sync_copy(beta_hbm, bv)
        gs = [gv[pl.ds(j * L, L)] for j in range(D // L)]
        bs = [bv[pl.ds(j * L, L)] for j in range(D // L)]

        def row_body(r, carry):
            vs = [rows_v[r, pl.ds(j * L, L)] for j in range(D // L)]
            s = vs[0] + vs[1] + (vs[2] + vs[3])
            ss = (vs[0] * vs[0] + vs[1] * vs[1]) + (vs[2] * vs[2] + vs[3] * vs[3])
            mean = _lane_sum(s) * jnp.float32(1.0 / D)
            var = _lane_sum(ss) * jnp.float32(1.0 / D) - mean * mean
            scale = _rsqrt_vec(var + jnp.float32(1e-5))
            for j in range(D // L):
                w = (vs[j] - mean) * scale * gs[j] + bs[j]
                rows_v[r, pl.ds(j * L, L)] = w
            return carry

        def chunk_body(g, carry):
            base = w_base + g * C
            pltpu.sync_copy(idx_hbm.at[pl.ds(base, C)], idx_v)
            pltpu.async_copy(table_hbm.at[idx_v], rows_v, sem).wait()
            lax.fori_loop(0, C, row_body, 0, unroll=2)
            pltpu.sync_copy(rows_v, out_hbm.at[pl.ds(base, C)])
            return carry

        lax.fori_loop(0, n_chunks, chunk_body, 0)

    return sc_fn


def kernel(x, table, gamma, beta):
    b, s = x.shape
    v, d = table.shape
    assert d == D
    xf = x.reshape(-1).astype(jnp.int32)
    out = _make_sc_call(b * s)(xf, table, gamma, beta)
    return out.reshape(b, s, d)
